# Initial kernel scaffold; baseline (speedup 1.0000x reference)
import jax
import jax.numpy as jnp
from jax.experimental import pallas as pl


def kernel(x, edge_index, W1, b1, W2, b2):
    raise NotImplementedError("write your pallas kernel here")


# speedup vs baseline: 7.6246x; 7.6246x over previous
import functools

import jax
import jax.numpy as jnp
from jax import lax
from jax.experimental import pallas as pl
from jax.experimental.pallas import tpu as pltpu
from jax.experimental.pallas import tpu_sc as plsc

N = 10000
DX = 128
H = 16
C = 64
E = 320000

NC = 2
NS = 16
B = 128
K = 80
KIDX = 84
EPAD = NC * NS * KIDX * B
DUMMY = N
R = 10112
ZROWS = R // NS

def _sc_body(y_hbm, src_hbm, dst_hbm, out_hbm,
             src_v, dst_v, rows_v, zw_v, agg_sh, sem_ga, sem_gb):
    c = lax.axis_index("c")
    s = lax.axis_index("s")

    pltpu.sync_copy(src_hbm.at[c, s], src_v)
    pltpu.sync_copy(dst_hbm.at[c, s], dst_v)

    def _zero(i, carry):
        zw_v[i, :] = jnp.zeros((16,), jnp.float32)
        return carry
    lax.fori_loop(0, ZROWS, _zero, 0)
    pltpu.sync_copy(zw_v, agg_sh.at[pl.ds(s * ZROWS, ZROWS)])
    plsc.subcore_barrier()

    def _g_start(chunk, buf, sem):
        pltpu.async_copy(y_hbm.at[src_v.at[chunk]], rows_v.at[buf], sem)

    def _g_wait(chunk, buf, sem):
        pltpu.make_async_copy(
            y_hbm.at[src_v.at[chunk]], rows_v.at[buf], sem).wait()

    for b in range(4):
        _g_start(b, b, sem_ga)

    def _phase(base, bufo, sem_cur, sem_nxt):
        for b in range(4):
            _g_wait(base + b, bufo + b, sem_cur)
            _g_start(base + 4 + b, (bufo + 4) % 8 + b, sem_nxt)
        for b in range(4):
            pltpu.sync_copy(
                rows_v.at[bufo + b], agg_sh.at[dst_v.at[base + b]], add=True)

    def _group(i, carry):
        _phase(8 * i, 0, sem_ga, sem_gb)
        _phase(8 * i + 4, 4, sem_gb, sem_ga)
        return carry
    lax.fori_loop(0, K // 8, _group, 0)

    for b in range(4):
        _g_wait(K + b, b, sem_ga)
    plsc.subcore_barrier()

    pltpu.sync_copy(agg_sh.at[pl.ds(s * ZROWS, ZROWS)], zw_v)
    pltpu.sync_copy(zw_v, out_hbm.at[c, pl.ds(s * ZROWS, ZROWS)])


@functools.cache
def _sc_edge_aggregate_fn():
    mesh = plsc.VectorSubcoreMesh(
        core_axis_name="c", subcore_axis_name="s",
        num_cores=NC, num_subcores=NS)
    return pl.kernel(
        _sc_body,
        out_type=jax.ShapeDtypeStruct((NC, R, H), jnp.float32),
        mesh=mesh,
        scratch_types=[
            pltpu.VMEM((KIDX, B), jnp.int32),
            pltpu.VMEM((KIDX, B), jnp.int32),
            pltpu.VMEM((8, B, H), jnp.float32),
            pltpu.VMEM((ZROWS, H), jnp.float32),
            pltpu.VMEM_SHARED((R, H), jnp.float32),
            pltpu.SemaphoreType.DMA,
            pltpu.SemaphoreType.DMA,
        ],
        compiler_params=pltpu.CompilerParams(use_tc_tiling_on_sc=False),
    )


def _sc_edge_aggregate(y, src4, dst4):
    return _sc_edge_aggregate_fn()(y, src4, dst4)


_ROWBLK = 2000


def _mm1_body(x_ref, w_ref, o_ref):
    o_ref[...] = jnp.dot(x_ref[...], w_ref[...],
                         preferred_element_type=jnp.float32)


def _project(x, w1):
    return pl.pallas_call(
        _mm1_body,
        grid=(N // _ROWBLK,),
        in_specs=[
            pl.BlockSpec((_ROWBLK, DX), lambda i: (i, 0)),
            pl.BlockSpec((DX, H), lambda i: (0, 0)),
        ],
        out_specs=pl.BlockSpec((_ROWBLK, H), lambda i: (i, 0)),
        out_shape=jax.ShapeDtypeStruct((N, H), jnp.float32),
    )(x, w1)


def _combine_body(y_ref, a0_ref, a1_ref, b_ref, o_ref):
    o_ref[...] = jnp.maximum(
        y_ref[...] + a0_ref[...] + a1_ref[...] + b_ref[...], 0.0)


def _combine_relu(y, a0, a1, b1):
    return pl.pallas_call(
        _combine_body,
        out_shape=jax.ShapeDtypeStruct((N, H), jnp.float32),
    )(y, a0, a1, b1)


def _layer2_body(h_ref, a0_ref, a1_ref, w_ref, b_ref, o_ref):
    z = jnp.dot(h_ref[...] + a0_ref[...] + a1_ref[...], w_ref[...],
                preferred_element_type=jnp.float32) + b_ref[...]
    m = jnp.max(z, axis=1, keepdims=True)
    e = z - m
    o_ref[...] = e - jnp.log(jnp.sum(jnp.exp(e), axis=1, keepdims=True))


def _layer2(h1, a0, a1, w2, b2):
    return pl.pallas_call(
        _layer2_body,
        grid=(N // _ROWBLK,),
        in_specs=[
            pl.BlockSpec((_ROWBLK, H), lambda i: (i, 0)),
            pl.BlockSpec((_ROWBLK, H), lambda i: (i, 0)),
            pl.BlockSpec((_ROWBLK, H), lambda i: (i, 0)),
            pl.BlockSpec((H, C), lambda i: (0, 0)),
            pl.BlockSpec((1, C), lambda i: (0, 0)),
        ],
        out_specs=pl.BlockSpec((_ROWBLK, C), lambda i: (i, 0)),
        out_shape=jax.ShapeDtypeStruct((N, C), jnp.float32),
    )(h1, a0, a1, w2, b2)


def kernel(x, edge_index, W1, b1, W2, b2):
    src = edge_index[0].astype(jnp.int32)
    dst = edge_index[1].astype(jnp.int32)
    pad = NC * NS * K * B - E
    srcp = jnp.concatenate([src, jnp.zeros((pad,), jnp.int32)])
    dstp = jnp.concatenate([dst, jnp.full((pad,), DUMMY, jnp.int32)])
    src4 = jnp.concatenate(
        [srcp.reshape(NC, NS, K, B),
         jnp.zeros((NC, NS, KIDX - K, B), jnp.int32)], axis=2)
    dst4 = jnp.concatenate(
        [dstp.reshape(NC, NS, K, B),
         jnp.full((NC, NS, KIDX - K, B), DUMMY, jnp.int32)], axis=2)

    y = _project(x, W1)
    agg1 = _sc_edge_aggregate(y, src4, dst4)
    h1 = _combine_relu(y, agg1[0, :N], agg1[1, :N], b1.reshape(1, H))
    agg2 = _sc_edge_aggregate(h1, src4, dst4)
    return _layer2(h1, agg2[0, :N], agg2[1, :N], W2, b2.reshape(1, C))
